# baseline (speedup 1.0000x reference)
import functools

import jax
import jax.numpy as jnp
from jax import lax
from jax.experimental import pallas as pl
from jax.experimental.pallas import tpu as pltpu
from jax.experimental.pallas import tpu_sc as plsc

_N = 10000
_E = 320000
_XD = 128
_HID = 128
_NCLS = 40

_NC = 2
_NS = 16
_NW = _NC * _NS
_K = 128
_NCHT = _E // _K
_T = 79
_NPAD = _N + 8

_RPS = _N // _NS
_ZCH = 125
_RBIG = 640
_RLAST = _N - 15 * _RBIG

_BN = 1000


def _worker_chunks(wid):
    qw = 78 * wid + jnp.minimum(wid, 4)
    cw = jnp.where(wid < 4, 79, 78)
    return qw, cw


def _fill(buf, val):
    v = jnp.full((16,), val, buf.dtype)

    @pl.loop(0, _K, step=16)
    def _(g):
        buf[pl.ds(g, 16)] = v


def _pre_body(x_ref, w1_ref, b1_ref, feat_ref, hp_ref):
    h = jnp.dot(x_ref[...], w1_ref[...], preferred_element_type=jnp.float32)
    h = h + b1_ref[...]
    feat_ref[...] = h
    hp_ref[...] = jnp.maximum(h, 0.0)


def _sc_cnt_body(e_hbm, src_out, dst_out, cnt_out,
                 ones_v, zbuf, pbuf, ib0, ib1, ib2, acc_sh,
                 i0, i1, i2, s0, s1, s2):
    c = lax.axis_index("c")
    s = lax.axis_index("s")
    wid = c * _NS + s
    qw, cw = _worker_chunks(wid)
    base = pl.multiple_of(qw * _K, 8)

    @pl.when(wid < 4)
    def _():
        for row, out in ((0, src_out), (1, dst_out)):
            pltpu.sync_copy(e_hbm.at[row, pl.ds(base, 79 * _K)],
                            pbuf.at[pl.ds(0, 79 * _K)])
            pltpu.sync_copy(pbuf.at[pl.ds(0, 79 * _K)],
                            out.at[pl.ds(base, 79 * _K)])

    @pl.when(wid >= 4)
    def _():
        for row, out in ((0, src_out), (1, dst_out)):
            pltpu.sync_copy(e_hbm.at[row, pl.ds(base, 78 * _K)],
                            pbuf.at[pl.ds(0, 78 * _K)])
            pltpu.sync_copy(pbuf.at[pl.ds(0, 78 * _K)],
                            out.at[pl.ds(base, 78 * _K)])

    ov = jnp.ones((16,), jnp.float32)
    zv = jnp.zeros((16,), jnp.float32)

    @pl.loop(0, _K)
    def _(r):
        ones_v[r, pl.ds(0, 16)] = ov

    @pl.loop(0, _ZCH)
    def _(r):
        zbuf[r, pl.ds(0, 16)] = zv

    @pl.loop(0, _RPS, step=_ZCH)
    def _(r0):
        pltpu.sync_copy(zbuf, acc_sh.at[pl.ds(s * _RPS + r0, _ZCH), :])

    plsc.subcore_barrier()

    ibs = (ib0, ib1, ib2)
    isems = (i0, i1, i2)
    ssems = (s0, s1, s2)

    def load_idx(t, ib, sem):
        off = pl.multiple_of((qw + jnp.minimum(t, _T - 1)) * _K, 8)
        pltpu.async_copy(e_hbm.at[1, pl.ds(off, _K)], ib, sem)

    def wait_idx(ib, sem):
        pltpu.make_async_copy(e_hbm.at[1, pl.ds(0, _K)], ib, sem).wait()

    def start_sc(ib, sem):
        pltpu.async_copy(ones_v, acc_sh.at[ib], sem, add=True)

    def wait_sc(sem):
        pltpu.make_async_copy(ones_v, acc_sh.at[pl.ds(0, _K), :], sem).wait()

    def step(t, sl, first=False, last=False):
        r, r2 = sl, (sl + 2) % 3
        wait_idx(ibs[r], isems[r])

        @pl.when(t >= cw)
        def _():
            _fill(ibs[r], _N)

        start_sc(ibs[r], ssems[r])
        if not first:
            wait_sc(ssems[r2])
        if not last:
            load_idx(t + 2, ibs[r2], isems[r2])

    load_idx(jnp.int32(0), ib0, i0)
    load_idx(jnp.int32(1), ib1, i1)
    step(jnp.int32(0), 0, first=True)

    @pl.loop(0, 25)
    def _(j3):
        t = 3 * j3 + 1
        step(t, 1)
        step(t + 1, 2)
        step(t + 2, 0)

    step(jnp.int32(76), 1)
    step(jnp.int32(77), 2)
    step(jnp.int32(78), 0, last=True)
    wait_idx(ibs[1], isems[1])
    wait_sc(ssems[0])

    plsc.subcore_barrier()

    pltpu.sync_copy(acc_sh.at[pl.ds(s * _RPS, _RPS), :],
                    cnt_out.at[c, pl.ds(s * _RPS, _RPS), :])


def _sc_agg_body(src_hbm, dst_hbm, hp_hbm, out_hbm,
                 sb0, sb1, sb2, db0, db1, db2, rows0, rows1, rows2, acc_sh,
                 i0, i1, i2, g0, g1, g2, s0, s1, s2):
    c = lax.axis_index("c")
    s = lax.axis_index("s")
    wid = c * _NS + s
    qw, cw = _worker_chunks(wid)

    row0 = s * _RBIG
    zv = jnp.zeros((16,), jnp.float32)

    @pl.loop(0, _K)
    def _(r):
        @pl.loop(0, _HID, step=16)
        def _(c0):
            rows0[r, pl.ds(c0, 16)] = zv

    @pl.when(s < 15)
    def _():
        @pl.loop(0, _RBIG, step=_K)
        def _(r0):
            pltpu.sync_copy(rows0, acc_sh.at[pl.ds(row0 + r0, _K), :])

    @pl.when(s == 15)
    def _():
        @pl.loop(0, _RLAST - 16, step=_K)
        def _(r0):
            pltpu.sync_copy(rows0, acc_sh.at[pl.ds(row0 + r0, _K), :])

        pltpu.sync_copy(rows0.at[pl.ds(0, 16 + 8), :],
                        acc_sh.at[pl.ds(row0 + _RLAST - 16, 16 + 8), :])

    plsc.subcore_barrier()

    sbs = (sb0, sb1, sb2)
    dbs = (db0, db1, db2)
    bufs = (rows0, rows1, rows2)
    isems = (i0, i1, i2)
    gsems = (g0, g1, g2)
    ssems = (s0, s1, s2)

    def load_idx(t, sb, db, sem):
        off = pl.multiple_of((qw + jnp.minimum(t, _T - 1)) * _K, 8)
        pltpu.async_copy(src_hbm.at[pl.ds(off, _K)], sb, sem)
        pltpu.async_copy(dst_hbm.at[pl.ds(off, _K)], db, sem)

    def wait_idx(sb, db, sem):
        pltpu.make_async_copy(src_hbm.at[pl.ds(0, _K)], sb, sem).wait()
        pltpu.make_async_copy(dst_hbm.at[pl.ds(0, _K)], db, sem).wait()

    def start_g(sb, buf, sem):
        pltpu.async_copy(hp_hbm.at[sb], buf, sem)

    def wait_g(buf, sem):
        pltpu.make_async_copy(hp_hbm.at[pl.ds(0, _K), :], buf, sem).wait()

    def start_sc(buf, db, sem):
        pltpu.async_copy(buf, acc_sh.at[db], sem, add=True)

    def wait_sc(buf, sem):
        pltpu.make_async_copy(buf, acc_sh.at[pl.ds(0, _K), :], sem).wait()

    def step(t, sl, first=False, last=False):
        r, r1, r2 = sl, (sl + 1) % 3, (sl + 2) % 3
        if not last:
            wait_idx(sbs[r1], dbs[r1], isems[r1])

            @pl.when(t + 1 >= cw)
            def _():
                _fill(sbs[r1], 0)
                _fill(dbs[r1], _N)

        if not first:
            wait_sc(bufs[r2], ssems[r2])
        wait_g(bufs[r], gsems[r])
        if not last:
            start_g(sbs[r1], bufs[r1], gsems[r1])
        start_sc(bufs[r], dbs[r], ssems[r])
        if not last:
            load_idx(t + 2, sbs[r2], dbs[r2], isems[r2])

    load_idx(jnp.int32(0), sb0, db0, i0)
    wait_idx(sb0, db0, i0)
    load_idx(jnp.int32(1), sb1, db1, i1)
    start_g(sb0, rows0, g0)
    step(jnp.int32(0), 0, first=True)

    @pl.loop(0, 25)
    def _(j3):
        t = 3 * j3 + 1
        step(t, 1)
        step(t + 1, 2)
        step(t + 2, 0)

    step(jnp.int32(76), 1)
    step(jnp.int32(77), 2)
    wait_idx(sbs[1], dbs[1], isems[1])
    wait_sc(bufs[2], ssems[2])
    wait_g(bufs[0], gsems[0])
    start_sc(bufs[0], dbs[0], ssems[0])
    wait_sc(bufs[0], ssems[0])

    plsc.subcore_barrier()

    @pl.when(s < 15)
    def _():
        pltpu.sync_copy(acc_sh.at[pl.ds(row0, _RBIG), :],
                        out_hbm.at[c, pl.ds(row0, _RBIG), :])

    @pl.when(s == 15)
    def _():
        pltpu.sync_copy(acc_sh.at[pl.ds(row0, _RLAST), :],
                        out_hbm.at[c, pl.ds(row0, _RLAST), :])


def _post_body(part_ref, cnt_ref, hp_ref, wl_ref, bl_ref, wr_ref, wn_ref,
               out_ref, of_ref):
    agg = part_ref[0] + part_ref[1]
    cnt = cnt_ref[0, :, :1] + cnt_ref[1, :, :1]
    mean = agg / jnp.maximum(cnt, 1.0)
    hr = hp_ref[...]
    of = jnp.dot(mean, wl_ref[...], preferred_element_type=jnp.float32)
    of = of + bl_ref[...]
    of = of + jnp.dot(hr, wr_ref[...], preferred_element_type=jnp.float32)
    of_ref[...] = of
    nrm = jnp.sqrt(jnp.sum(of * of, axis=1, keepdims=True))
    xn = of / jnp.maximum(nrm, 1e-12)
    w = wn_ref[...]
    wnrm = jnp.sqrt(jnp.sum(w * w, axis=0, keepdims=True))
    wn = w / jnp.maximum(wnrm, 1e-12)
    out_ref[...] = 10.0 * jnp.dot(xn, wn, preferred_element_type=jnp.float32)


@jax.jit
def _run(x, e, W1, b1, Wl, bl, Wr, Wn):
    mesh = plsc.VectorSubcoreMesh(core_axis_name="c", subcore_axis_name="s")

    src1d, dst1d, cnt = pl.kernel(
        _sc_cnt_body,
        out_type=[
            jax.ShapeDtypeStruct((_E,), jnp.int32),
            jax.ShapeDtypeStruct((_E,), jnp.int32),
            jax.ShapeDtypeStruct((_NC, _N, 16), jnp.float32),
        ],
        mesh=mesh,
        compiler_params=pltpu.CompilerParams(use_tc_tiling_on_sc=False),
        scratch_types=[
            pltpu.VMEM((_K, 16), jnp.float32),
            pltpu.VMEM((_ZCH, 16), jnp.float32),
            pltpu.VMEM((79 * _K,), jnp.int32),
            pltpu.VMEM((_K,), jnp.int32),
            pltpu.VMEM((_K,), jnp.int32),
            pltpu.VMEM((_K,), jnp.int32),
            pltpu.VMEM_SHARED((_NPAD, 16), jnp.float32),
        ] + [pltpu.SemaphoreType.DMA] * 6,
    )(e)

    feat, hp = pl.pallas_call(
        _pre_body,
        grid=(_N // _BN,),
        in_specs=[
            pl.BlockSpec((_BN, _XD), lambda i: (i, 0)),
            pl.BlockSpec((_XD, _HID), lambda i: (0, 0)),
            pl.BlockSpec((1, _HID), lambda i: (0, 0)),
        ],
        out_specs=[
            pl.BlockSpec((_BN, _HID), lambda i: (i, 0)),
            pl.BlockSpec((_BN, _HID), lambda i: (i, 0)),
        ],
        out_shape=[
            jax.ShapeDtypeStruct((_N, _HID), jnp.float32),
            jax.ShapeDtypeStruct((_N, _HID), jnp.float32),
        ],
    )(x, W1, b1.reshape(1, _HID))

    partials = pl.kernel(
        _sc_agg_body,
        out_type=jax.ShapeDtypeStruct((_NC, _N, _HID), jnp.float32),
        mesh=mesh,
        compiler_params=pltpu.CompilerParams(use_tc_tiling_on_sc=True),
        scratch_types=[
            pltpu.VMEM((_K,), jnp.int32),
            pltpu.VMEM((_K,), jnp.int32),
            pltpu.VMEM((_K,), jnp.int32),
            pltpu.VMEM((_K,), jnp.int32),
            pltpu.VMEM((_K,), jnp.int32),
            pltpu.VMEM((_K,), jnp.int32),
            pltpu.VMEM((_K, _HID), jnp.float32),
            pltpu.VMEM((_K, _HID), jnp.float32),
            pltpu.VMEM((_K, _HID), jnp.float32),
            pltpu.VMEM_SHARED((_NPAD, _HID), jnp.float32),
        ] + [pltpu.SemaphoreType.DMA] * 9,
    )(src1d, dst1d, hp)

    out, out_feat = pl.pallas_call(
        _post_body,
        grid=(_N // _BN,),
        in_specs=[
            pl.BlockSpec((_NC, _BN, _HID), lambda i: (0, i, 0)),
            pl.BlockSpec((_NC, _BN, 16), lambda i: (0, i, 0)),
            pl.BlockSpec((_BN, _HID), lambda i: (i, 0)),
            pl.BlockSpec((_HID, _HID), lambda i: (0, 0)),
            pl.BlockSpec((1, _HID), lambda i: (0, 0)),
            pl.BlockSpec((_HID, _HID), lambda i: (0, 0)),
            pl.BlockSpec((_HID, _NCLS), lambda i: (0, 0)),
        ],
        out_specs=[
            pl.BlockSpec((_BN, _NCLS), lambda i: (i, 0)),
            pl.BlockSpec((_BN, _HID), lambda i: (i, 0)),
        ],
        out_shape=[
            jax.ShapeDtypeStruct((_N, _NCLS), jnp.float32),
            jax.ShapeDtypeStruct((_N, _HID), jnp.float32),
        ],
    )(partials, cnt, hp, Wl, bl.reshape(1, _HID), Wr, Wn)

    return out, feat, out_feat


def kernel(x, edge_index, W1, b1, Wl, bl, Wr, Wn):
    return _run(x, edge_index.astype(jnp.int32), W1, b1, Wl, bl, Wr, Wn)


# speedup vs baseline: 1.4622x; 1.4622x over previous
import functools

import jax
import jax.numpy as jnp
from jax import lax
from jax.experimental import pallas as pl
from jax.experimental.pallas import tpu as pltpu
from jax.experimental.pallas import tpu_sc as plsc

_N = 10000
_E = 320000
_XD = 128
_HID = 128
_NCLS = 40

_NC = 2
_NS = 16
_NW = _NC * _NS
_K = 128
_NCHT = _E // _K
_T = 79
_NPAD = _N + 8

_RPS = _N // _NS
_KF = 80
_EPW = _E // _NW
_NCHF = _EPW // _KF
_ZCH = 125
_RBIG = 640
_RLAST = _N - 15 * _RBIG

_BN = 1000


def _worker_chunks(wid):
    qw = 78 * wid + jnp.minimum(wid, 4)
    cw = jnp.where(wid < 4, 79, 78)
    return qw, cw


def _fill(buf, val):
    v = jnp.full((16,), val, buf.dtype)

    @pl.loop(0, _K, step=16)
    def _(g):
        buf[pl.ds(g, 16)] = v


def _pre_body(x_ref, w1_ref, b1_ref, feat_ref, hp_ref):
    h = jnp.dot(x_ref[...], w1_ref[...], preferred_element_type=jnp.float32)
    h = h + b1_ref[...]
    feat_ref[...] = h
    hp_ref[...] = jnp.maximum(h, 0.0)


def _sc_cnt_body(e_hbm, src_out, dst_out, cnt_out,
                 ones_v, zbuf, pbuf, ib0, ib1, ib2, acc_sh,
                 i0, i1, i2, s0, s1, s2):
    c = lax.axis_index("c")
    s = lax.axis_index("s")
    wid = c * _NS + s
    qw, cw = _worker_chunks(wid)
    base = pl.multiple_of(qw * _K, 8)

    @pl.when(wid < 4)
    def _():
        for row, out in ((0, src_out), (1, dst_out)):
            pltpu.sync_copy(e_hbm.at[row, pl.ds(base, 79 * _K)],
                            pbuf.at[pl.ds(0, 79 * _K)])
            pltpu.sync_copy(pbuf.at[pl.ds(0, 79 * _K)],
                            out.at[pl.ds(base, 79 * _K)])

    @pl.when(wid >= 4)
    def _():
        for row, out in ((0, src_out), (1, dst_out)):
            pltpu.sync_copy(e_hbm.at[row, pl.ds(base, 78 * _K)],
                            pbuf.at[pl.ds(0, 78 * _K)])
            pltpu.sync_copy(pbuf.at[pl.ds(0, 78 * _K)],
                            out.at[pl.ds(base, 78 * _K)])

    ov = jnp.ones((16,), jnp.float32)
    zv = jnp.zeros((16,), jnp.float32)

    @pl.loop(0, _K)
    def _(r):
        ones_v[r, pl.ds(0, 16)] = ov

    @pl.loop(0, _ZCH)
    def _(r):
        zbuf[r, pl.ds(0, 16)] = zv

    @pl.loop(0, _RPS, step=_ZCH)
    def _(r0):
        pltpu.sync_copy(zbuf, acc_sh.at[pl.ds(s * _RPS + r0, _ZCH), :])

    plsc.subcore_barrier()

    ibs = (ib0, ib1, ib2)
    isems = (i0, i1, i2)
    ssems = (s0, s1, s2)

    def load_idx(t, ib, sem):
        off = pl.multiple_of((qw + jnp.minimum(t, _T - 1)) * _K, 8)
        pltpu.async_copy(e_hbm.at[1, pl.ds(off, _K)], ib, sem)

    def wait_idx(ib, sem):
        pltpu.make_async_copy(e_hbm.at[1, pl.ds(0, _K)], ib, sem).wait()

    def start_sc(ib, sem):
        pltpu.async_copy(ones_v, acc_sh.at[ib], sem, add=True)

    def wait_sc(sem):
        pltpu.make_async_copy(ones_v, acc_sh.at[pl.ds(0, _K), :], sem).wait()

    def step(t, sl, first=False, last=False):
        r, r2 = sl, (sl + 2) % 3
        wait_idx(ibs[r], isems[r])

        @pl.when(t >= cw)
        def _():
            _fill(ibs[r], _N)

        start_sc(ibs[r], ssems[r])
        if not first:
            wait_sc(ssems[r2])
        if not last:
            load_idx(t + 2, ibs[r2], isems[r2])

    load_idx(jnp.int32(0), ib0, i0)
    load_idx(jnp.int32(1), ib1, i1)
    step(jnp.int32(0), 0, first=True)

    @pl.loop(0, 25)
    def _(j3):
        t = 3 * j3 + 1
        step(t, 1)
        step(t + 1, 2)
        step(t + 2, 0)

    step(jnp.int32(76), 1)
    step(jnp.int32(77), 2)
    step(jnp.int32(78), 0, last=True)
    wait_idx(ibs[1], isems[1])
    wait_sc(ssems[0])

    plsc.subcore_barrier()

    pltpu.sync_copy(acc_sh.at[pl.ds(s * _RPS, _RPS), :],
                    cnt_out.at[c, pl.ds(s * _RPS, _RPS), :])


def _sc_agg_body(src_hbm, dst_hbm, hp_hbm, out_hbm,
                 sb0, sb1, sb2, db0, db1, db2, rows0, rows1, rows2, acc_sh,
                 i0, i1, i2, g0, g1, g2, s0, s1, s2):
    c = lax.axis_index("c")
    s = lax.axis_index("s")
    wid = c * _NS + s
    base0 = wid * _EPW

    row0 = s * _RBIG
    zv = jnp.zeros((16,), jnp.float32)

    @pl.loop(0, _KF)
    def _(r):
        @pl.loop(0, _HID, step=16)
        def _(c0):
            rows0[r, pl.ds(c0, 16)] = zv

    @pl.when(s < 15)
    def _():
        @pl.loop(0, _RBIG, step=_KF)
        def _(r0):
            pltpu.sync_copy(rows0, acc_sh.at[pl.ds(row0 + r0, _KF), :])

    @pl.when(s == 15)
    def _():
        @pl.loop(0, _RLAST, step=_KF)
        def _(r0):
            pltpu.sync_copy(rows0, acc_sh.at[pl.ds(row0 + r0, _KF), :])

    plsc.subcore_barrier()

    sbs = (sb0, sb1, sb2)
    dbs = (db0, db1, db2)
    bufs = (rows0, rows1, rows2)
    isems = (i0, i1, i2)
    gsems = (g0, g1, g2)
    ssems = (s0, s1, s2)

    def load_idx(t, sb, db, sem):
        off = pl.multiple_of(base0 + jnp.minimum(t, _NCHF - 1) * _KF, 8)
        pltpu.async_copy(src_hbm.at[pl.ds(off, _KF)], sb, sem)
        pltpu.async_copy(dst_hbm.at[pl.ds(off, _KF)], db, sem)

    def wait_idx(sb, db, sem):
        pltpu.make_async_copy(src_hbm.at[pl.ds(0, _KF)], sb, sem).wait()
        pltpu.make_async_copy(dst_hbm.at[pl.ds(0, _KF)], db, sem).wait()

    def start_g(sb, buf, sem):
        pltpu.async_copy(hp_hbm.at[sb], buf, sem)

    def wait_g(buf, sem):
        pltpu.make_async_copy(hp_hbm.at[pl.ds(0, _KF), :], buf, sem).wait()

    def start_sc(buf, db, sem):
        pltpu.async_copy(buf, acc_sh.at[db], sem, add=True)

    def wait_sc(buf, sem):
        pltpu.make_async_copy(buf, acc_sh.at[pl.ds(0, _KF), :], sem).wait()

    def step(t, sl, first=False, last=False):
        r, r1, r2 = sl, (sl + 1) % 3, (sl + 2) % 3
        if not last:
            wait_idx(sbs[r1], dbs[r1], isems[r1])
        if not first:
            wait_sc(bufs[r2], ssems[r2])
        wait_g(bufs[r], gsems[r])
        if not last:
            start_g(sbs[r1], bufs[r1], gsems[r1])
        start_sc(bufs[r], dbs[r], ssems[r])
        if not last:
            load_idx(t + 2, sbs[r2], dbs[r2], isems[r2])

    load_idx(0, sb0, db0, i0)
    wait_idx(sb0, db0, i0)
    load_idx(1, sb1, db1, i1)
    start_g(sb0, rows0, g0)
    step(jnp.int32(0), 0, first=True)

    @pl.loop(0, (_NCHF - 2) // 3)
    def _(j3):
        t = 3 * j3 + 1
        step(t, 1)
        step(t + 1, 2)
        step(t + 2, 0)

    wait_idx(sbs[2], dbs[2], isems[2])
    wait_sc(bufs[0], ssems[0])
    wait_g(bufs[1], gsems[1])
    start_sc(bufs[1], dbs[1], ssems[1])
    wait_sc(bufs[1], ssems[1])

    plsc.subcore_barrier()

    @pl.when(s < 15)
    def _():
        pltpu.sync_copy(acc_sh.at[pl.ds(row0, _RBIG), :],
                        out_hbm.at[c, pl.ds(row0, _RBIG), :])

    @pl.when(s == 15)
    def _():
        pltpu.sync_copy(acc_sh.at[pl.ds(row0, _RLAST), :],
                        out_hbm.at[c, pl.ds(row0, _RLAST), :])


def _post_body(part_ref, cnt_ref, hp_ref, wl_ref, bl_ref, wr_ref, wn_ref,
               out_ref, of_ref):
    agg = part_ref[0] + part_ref[1]
    cnt = cnt_ref[0, :, :1] + cnt_ref[1, :, :1]
    mean = agg / jnp.maximum(cnt, 1.0)
    hr = hp_ref[...]
    of = jnp.dot(mean, wl_ref[...], preferred_element_type=jnp.float32)
    of = of + bl_ref[...]
    of = of + jnp.dot(hr, wr_ref[...], preferred_element_type=jnp.float32)
    of_ref[...] = of
    nrm = jnp.sqrt(jnp.sum(of * of, axis=1, keepdims=True))
    xn = of / jnp.maximum(nrm, 1e-12)
    w = wn_ref[...]
    wnrm = jnp.sqrt(jnp.sum(w * w, axis=0, keepdims=True))
    wn = w / jnp.maximum(wnrm, 1e-12)
    out_ref[...] = 10.0 * jnp.dot(xn, wn, preferred_element_type=jnp.float32)


@jax.jit
def _run(x, e, W1, b1, Wl, bl, Wr, Wn):
    mesh = plsc.VectorSubcoreMesh(core_axis_name="c", subcore_axis_name="s")

    src1d, dst1d, cnt = pl.kernel(
        _sc_cnt_body,
        out_type=[
            jax.ShapeDtypeStruct((_E,), jnp.int32),
            jax.ShapeDtypeStruct((_E,), jnp.int32),
            jax.ShapeDtypeStruct((_NC, _N, 16), jnp.float32),
        ],
        mesh=mesh,
        compiler_params=pltpu.CompilerParams(use_tc_tiling_on_sc=False),
        scratch_types=[
            pltpu.VMEM((_K, 16), jnp.float32),
            pltpu.VMEM((_ZCH, 16), jnp.float32),
            pltpu.VMEM((79 * _K,), jnp.int32),
            pltpu.VMEM((_K,), jnp.int32),
            pltpu.VMEM((_K,), jnp.int32),
            pltpu.VMEM((_K,), jnp.int32),
            pltpu.VMEM_SHARED((_NPAD, 16), jnp.float32),
        ] + [pltpu.SemaphoreType.DMA] * 6,
    )(e)

    feat, hp = pl.pallas_call(
        _pre_body,
        grid=(_N // _BN,),
        in_specs=[
            pl.BlockSpec((_BN, _XD), lambda i: (i, 0)),
            pl.BlockSpec((_XD, _HID), lambda i: (0, 0)),
            pl.BlockSpec((1, _HID), lambda i: (0, 0)),
        ],
        out_specs=[
            pl.BlockSpec((_BN, _HID), lambda i: (i, 0)),
            pl.BlockSpec((_BN, _HID), lambda i: (i, 0)),
        ],
        out_shape=[
            jax.ShapeDtypeStruct((_N, _HID), jnp.float32),
            jax.ShapeDtypeStruct((_N, _HID), jnp.float32),
        ],
    )(x, W1, b1.reshape(1, _HID))

    partials = pl.kernel(
        _sc_agg_body,
        out_type=jax.ShapeDtypeStruct((_NC, _N, _HID), jnp.float32),
        mesh=mesh,
        compiler_params=pltpu.CompilerParams(use_tc_tiling_on_sc=True),
        scratch_types=[
            pltpu.VMEM((_KF,), jnp.int32),
            pltpu.VMEM((_KF,), jnp.int32),
            pltpu.VMEM((_KF,), jnp.int32),
            pltpu.VMEM((_KF,), jnp.int32),
            pltpu.VMEM((_KF,), jnp.int32),
            pltpu.VMEM((_KF,), jnp.int32),
            pltpu.VMEM((_KF, _HID), jnp.float32),
            pltpu.VMEM((_KF, _HID), jnp.float32),
            pltpu.VMEM((_KF, _HID), jnp.float32),
            pltpu.VMEM_SHARED((_N, _HID), jnp.float32),
        ] + [pltpu.SemaphoreType.DMA] * 9,
    )(src1d, dst1d, hp)

    out, out_feat = pl.pallas_call(
        _post_body,
        grid=(_N // _BN,),
        in_specs=[
            pl.BlockSpec((_NC, _BN, _HID), lambda i: (0, i, 0)),
            pl.BlockSpec((_NC, _BN, 16), lambda i: (0, i, 0)),
            pl.BlockSpec((_BN, _HID), lambda i: (i, 0)),
            pl.BlockSpec((_HID, _HID), lambda i: (0, 0)),
            pl.BlockSpec((1, _HID), lambda i: (0, 0)),
            pl.BlockSpec((_HID, _HID), lambda i: (0, 0)),
            pl.BlockSpec((_HID, _NCLS), lambda i: (0, 0)),
        ],
        out_specs=[
            pl.BlockSpec((_BN, _NCLS), lambda i: (i, 0)),
            pl.BlockSpec((_BN, _HID), lambda i: (i, 0)),
        ],
        out_shape=[
            jax.ShapeDtypeStruct((_N, _NCLS), jnp.float32),
            jax.ShapeDtypeStruct((_N, _HID), jnp.float32),
        ],
    )(partials, cnt, hp, Wl, bl.reshape(1, _HID), Wr, Wn)

    return out, feat, out_feat


def kernel(x, edge_index, W1, b1, Wl, bl, Wr, Wn):
    return _run(x, edge_index.astype(jnp.int32), W1, b1, Wl, bl, Wr, Wn)


# speedup vs baseline: 1.4632x; 1.0007x over previous
import functools

import jax
import jax.numpy as jnp
from jax import lax
from jax.experimental import pallas as pl
from jax.experimental.pallas import tpu as pltpu
from jax.experimental.pallas import tpu_sc as plsc

_N = 10000
_E = 320000
_XD = 128
_HID = 128
_NCLS = 40

_NC = 2
_NS = 16
_NW = _NC * _NS
_K = 128
_NCHT = _E // _K
_T = 79
_NPAD = _N + 8

_RPS = _N // _NS
_KF = 80
_EPW = _E // _NW
_NCHF = _EPW // _KF
_ZCH = 125
_RBIG = 640
_RLAST = _N - 15 * _RBIG

_BN = 1000


def _worker_chunks(wid):
    qw = 78 * wid + jnp.minimum(wid, 4)
    cw = jnp.where(wid < 4, 79, 78)
    return qw, cw


def _fill(buf, val):
    v = jnp.full((16,), val, buf.dtype)

    @pl.loop(0, _K, step=16)
    def _(g):
        buf[pl.ds(g, 16)] = v


def _pre_body(x_ref, w1_ref, b1_ref, feat_ref, hp_ref):
    h = jnp.dot(x_ref[...], w1_ref[...], preferred_element_type=jnp.float32)
    h = h + b1_ref[...]
    feat_ref[...] = h
    hp_ref[...] = jnp.maximum(h, 0.0)


def _sc_cnt_body(e_hbm, src_out, dst_out, cnt_out,
                 ones_v, zbuf, pbuf, pbuf2, ib0, ib1, ib2, acc_sh,
                 i0, i1, i2, s0, s1, s2, psem):
    c = lax.axis_index("c")
    s = lax.axis_index("s")
    wid = c * _NS + s
    qw, cw = _worker_chunks(wid)
    base = pl.multiple_of(qw * _K, 8)

    def pass_leg(n, inbound):
        for row, buf, out in ((0, pbuf, src_out), (1, pbuf2, dst_out)):
            if inbound:
                pltpu.async_copy(e_hbm.at[row, pl.ds(base, n * _K)],
                                 buf.at[pl.ds(0, n * _K)], psem)
            else:
                pltpu.async_copy(buf.at[pl.ds(0, n * _K)],
                                 out.at[pl.ds(base, n * _K)], psem)

    def pass_wait(n):
        for buf in (pbuf, pbuf2):
            pltpu.make_async_copy(e_hbm.at[0, pl.ds(0, n * _K)],
                                  buf.at[pl.ds(0, n * _K)], psem).wait()

    @pl.when(wid < 4)
    def _():
        pass_leg(79, True)

    @pl.when(wid >= 4)
    def _():
        pass_leg(78, True)

    ov = jnp.ones((16,), jnp.float32)
    zv = jnp.zeros((16,), jnp.float32)

    @pl.loop(0, _K)
    def _(r):
        ones_v[r, pl.ds(0, 16)] = ov

    @pl.loop(0, _ZCH)
    def _(r):
        zbuf[r, pl.ds(0, 16)] = zv

    @pl.when(wid < 4)
    def _():
        pass_wait(79)
        pass_leg(79, False)

    @pl.when(wid >= 4)
    def _():
        pass_wait(78)
        pass_leg(78, False)

    @pl.loop(0, _RPS, step=_ZCH)
    def _(r0):
        pltpu.sync_copy(zbuf, acc_sh.at[pl.ds(s * _RPS + r0, _ZCH), :])

    @pl.when(wid < 4)
    def _():
        pass_wait(79)

    @pl.when(wid >= 4)
    def _():
        pass_wait(78)

    plsc.subcore_barrier()

    ibs = (ib0, ib1, ib2)
    isems = (i0, i1, i2)
    ssems = (s0, s1, s2)

    def load_idx(t, ib, sem):
        off = pl.multiple_of((qw + jnp.minimum(t, _T - 1)) * _K, 8)
        pltpu.async_copy(e_hbm.at[1, pl.ds(off, _K)], ib, sem)

    def wait_idx(ib, sem):
        pltpu.make_async_copy(e_hbm.at[1, pl.ds(0, _K)], ib, sem).wait()

    def start_sc(ib, sem):
        pltpu.async_copy(ones_v, acc_sh.at[ib], sem, add=True)

    def wait_sc(sem):
        pltpu.make_async_copy(ones_v, acc_sh.at[pl.ds(0, _K), :], sem).wait()

    def step(t, sl, first=False, last=False):
        r, r2 = sl, (sl + 2) % 3
        wait_idx(ibs[r], isems[r])

        @pl.when(t >= cw)
        def _():
            _fill(ibs[r], _N)

        start_sc(ibs[r], ssems[r])
        if not first:
            wait_sc(ssems[r2])
        if not last:
            load_idx(t + 2, ibs[r2], isems[r2])

    load_idx(jnp.int32(0), ib0, i0)
    load_idx(jnp.int32(1), ib1, i1)
    step(jnp.int32(0), 0, first=True)

    @pl.loop(0, 25)
    def _(j3):
        t = 3 * j3 + 1
        step(t, 1)
        step(t + 1, 2)
        step(t + 2, 0)

    step(jnp.int32(76), 1)
    step(jnp.int32(77), 2)
    step(jnp.int32(78), 0, last=True)
    wait_idx(ibs[1], isems[1])
    wait_sc(ssems[0])

    plsc.subcore_barrier()

    pltpu.sync_copy(acc_sh.at[pl.ds(s * _RPS, _RPS), :],
                    cnt_out.at[c, pl.ds(s * _RPS, _RPS), :])


def _sc_agg_body(src_hbm, dst_hbm, hp_hbm, out_hbm,
                 sb0, sb1, sb2, db0, db1, db2, rows0, rows1, rows2, acc_sh,
                 i0, i1, i2, g0, g1, g2, s0, s1, s2):
    c = lax.axis_index("c")
    s = lax.axis_index("s")
    wid = c * _NS + s
    base0 = wid * _EPW

    row0 = s * _RBIG
    zv = jnp.zeros((16,), jnp.float32)

    @pl.loop(0, _KF)
    def _(r):
        @pl.loop(0, _HID, step=16)
        def _(c0):
            rows0[r, pl.ds(c0, 16)] = zv

    @pl.when(s < 15)
    def _():
        @pl.loop(0, _RBIG, step=_KF)
        def _(r0):
            pltpu.sync_copy(rows0, acc_sh.at[pl.ds(row0 + r0, _KF), :])

    @pl.when(s == 15)
    def _():
        @pl.loop(0, _RLAST, step=_KF)
        def _(r0):
            pltpu.sync_copy(rows0, acc_sh.at[pl.ds(row0 + r0, _KF), :])

    plsc.subcore_barrier()

    sbs = (sb0, sb1, sb2)
    dbs = (db0, db1, db2)
    bufs = (rows0, rows1, rows2)
    isems = (i0, i1, i2)
    gsems = (g0, g1, g2)
    ssems = (s0, s1, s2)

    def load_idx(t, sb, db, sem):
        off = pl.multiple_of(base0 + jnp.minimum(t, _NCHF - 1) * _KF, 8)
        pltpu.async_copy(src_hbm.at[pl.ds(off, _KF)], sb, sem)
        pltpu.async_copy(dst_hbm.at[pl.ds(off, _KF)], db, sem)

    def wait_idx(sb, db, sem):
        pltpu.make_async_copy(src_hbm.at[pl.ds(0, _KF)], sb, sem).wait()
        pltpu.make_async_copy(dst_hbm.at[pl.ds(0, _KF)], db, sem).wait()

    def start_g(sb, buf, sem):
        pltpu.async_copy(hp_hbm.at[sb], buf, sem)

    def wait_g(buf, sem):
        pltpu.make_async_copy(hp_hbm.at[pl.ds(0, _KF), :], buf, sem).wait()

    def start_sc(buf, db, sem):
        pltpu.async_copy(buf, acc_sh.at[db], sem, add=True)

    def wait_sc(buf, sem):
        pltpu.make_async_copy(buf, acc_sh.at[pl.ds(0, _KF), :], sem).wait()

    def step(t, sl, first=False, last=False):
        r, r1, r2 = sl, (sl + 1) % 3, (sl + 2) % 3
        if not last:
            wait_idx(sbs[r1], dbs[r1], isems[r1])
        if not first:
            wait_sc(bufs[r2], ssems[r2])
        wait_g(bufs[r], gsems[r])
        if not last:
            start_g(sbs[r1], bufs[r1], gsems[r1])
        start_sc(bufs[r], dbs[r], ssems[r])
        if not last:
            load_idx(t + 2, sbs[r2], dbs[r2], isems[r2])

    load_idx(0, sb0, db0, i0)
    wait_idx(sb0, db0, i0)
    load_idx(1, sb1, db1, i1)
    start_g(sb0, rows0, g0)
    step(jnp.int32(0), 0, first=True)

    @pl.loop(0, (_NCHF - 2) // 3)
    def _(j3):
        t = 3 * j3 + 1
        step(t, 1)
        step(t + 1, 2)
        step(t + 2, 0)

    wait_idx(sbs[2], dbs[2], isems[2])
    wait_sc(bufs[0], ssems[0])
    wait_g(bufs[1], gsems[1])
    start_sc(bufs[1], dbs[1], ssems[1])
    wait_sc(bufs[1], ssems[1])

    plsc.subcore_barrier()

    @pl.when(s < 15)
    def _():
        pltpu.sync_copy(acc_sh.at[pl.ds(row0, _RBIG), :],
                        out_hbm.at[c, pl.ds(row0, _RBIG), :])

    @pl.when(s == 15)
    def _():
        pltpu.sync_copy(acc_sh.at[pl.ds(row0, _RLAST), :],
                        out_hbm.at[c, pl.ds(row0, _RLAST), :])


def _root_body(hp_ref, wr_ref, bl_ref, z_ref):
    z_ref[...] = (jnp.dot(hp_ref[...], wr_ref[...],
                          preferred_element_type=jnp.float32) + bl_ref[...])


def _post_body(part_ref, cnt_ref, z_ref, wl_ref, wn_ref,
               out_ref, of_ref):
    agg = part_ref[0] + part_ref[1]
    cnt = cnt_ref[0, :, :1] + cnt_ref[1, :, :1]
    mean = agg / jnp.maximum(cnt, 1.0)
    of = jnp.dot(mean, wl_ref[...], preferred_element_type=jnp.float32)
    of = of + z_ref[...]
    of_ref[...] = of
    nrm = jnp.sqrt(jnp.sum(of * of, axis=1, keepdims=True))
    xn = of / jnp.maximum(nrm, 1e-12)
    w = wn_ref[...]
    wnrm = jnp.sqrt(jnp.sum(w * w, axis=0, keepdims=True))
    wn = w / jnp.maximum(wnrm, 1e-12)
    out_ref[...] = 10.0 * jnp.dot(xn, wn, preferred_element_type=jnp.float32)


@jax.jit
def _run(x, e, W1, b1, Wl, bl, Wr, Wn):
    mesh = plsc.VectorSubcoreMesh(core_axis_name="c", subcore_axis_name="s")

    src1d, dst1d, cnt = pl.kernel(
        _sc_cnt_body,
        out_type=[
            jax.ShapeDtypeStruct((_E,), jnp.int32),
            jax.ShapeDtypeStruct((_E,), jnp.int32),
            jax.ShapeDtypeStruct((_NC, _N, 16), jnp.float32),
        ],
        mesh=mesh,
        compiler_params=pltpu.CompilerParams(use_tc_tiling_on_sc=False),
        scratch_types=[
            pltpu.VMEM((_K, 16), jnp.float32),
            pltpu.VMEM((_ZCH, 16), jnp.float32),
            pltpu.VMEM((79 * _K,), jnp.int32),
            pltpu.VMEM((79 * _K,), jnp.int32),
            pltpu.VMEM((_K,), jnp.int32),
            pltpu.VMEM((_K,), jnp.int32),
            pltpu.VMEM((_K,), jnp.int32),
            pltpu.VMEM_SHARED((_NPAD, 16), jnp.float32),
        ] + [pltpu.SemaphoreType.DMA] * 7,
    )(e)

    feat, hp = pl.pallas_call(
        _pre_body,
        grid=(_N // _BN,),
        in_specs=[
            pl.BlockSpec((_BN, _XD), lambda i: (i, 0)),
            pl.BlockSpec((_XD, _HID), lambda i: (0, 0)),
            pl.BlockSpec((1, _HID), lambda i: (0, 0)),
        ],
        out_specs=[
            pl.BlockSpec((_BN, _HID), lambda i: (i, 0)),
            pl.BlockSpec((_BN, _HID), lambda i: (i, 0)),
        ],
        out_shape=[
            jax.ShapeDtypeStruct((_N, _HID), jnp.float32),
            jax.ShapeDtypeStruct((_N, _HID), jnp.float32),
        ],
    )(x, W1, b1.reshape(1, _HID))

    partials = pl.kernel(
        _sc_agg_body,
        out_type=jax.ShapeDtypeStruct((_NC, _N, _HID), jnp.float32),
        mesh=mesh,
        compiler_params=pltpu.CompilerParams(use_tc_tiling_on_sc=True),
        scratch_types=[
            pltpu.VMEM((_KF,), jnp.int32),
            pltpu.VMEM((_KF,), jnp.int32),
            pltpu.VMEM((_KF,), jnp.int32),
            pltpu.VMEM((_KF,), jnp.int32),
            pltpu.VMEM((_KF,), jnp.int32),
            pltpu.VMEM((_KF,), jnp.int32),
            pltpu.VMEM((_KF, _HID), jnp.float32),
            pltpu.VMEM((_KF, _HID), jnp.float32),
            pltpu.VMEM((_KF, _HID), jnp.float32),
            pltpu.VMEM_SHARED((_N, _HID), jnp.float32),
        ] + [pltpu.SemaphoreType.DMA] * 9,
    )(src1d, dst1d, hp)

    z = pl.pallas_call(
        _root_body,
        grid=(_N // _BN,),
        in_specs=[
            pl.BlockSpec((_BN, _HID), lambda i: (i, 0)),
            pl.BlockSpec((_HID, _HID), lambda i: (0, 0)),
            pl.BlockSpec((1, _HID), lambda i: (0, 0)),
        ],
        out_specs=pl.BlockSpec((_BN, _HID), lambda i: (i, 0)),
        out_shape=jax.ShapeDtypeStruct((_N, _HID), jnp.float32),
    )(hp, Wr, bl.reshape(1, _HID))

    out, out_feat = pl.pallas_call(
        _post_body,
        grid=(_N // _BN,),
        in_specs=[
            pl.BlockSpec((_NC, _BN, _HID), lambda i: (0, i, 0)),
            pl.BlockSpec((_NC, _BN, 16), lambda i: (0, i, 0)),
            pl.BlockSpec((_BN, _HID), lambda i: (i, 0)),
            pl.BlockSpec((_HID, _HID), lambda i: (0, 0)),
            pl.BlockSpec((_HID, _NCLS), lambda i: (0, 0)),
        ],
        out_specs=[
            pl.BlockSpec((_BN, _NCLS), lambda i: (i, 0)),
            pl.BlockSpec((_BN, _HID), lambda i: (i, 0)),
        ],
        out_shape=[
            jax.ShapeDtypeStruct((_N, _NCLS), jnp.float32),
            jax.ShapeDtypeStruct((_N, _HID), jnp.float32),
        ],
    )(partials, cnt, z, Wl, Wn)

    return out, feat, out_feat


def kernel(x, edge_index, W1, b1, Wl, bl, Wr, Wn):
    return _run(x, edge_index.astype(jnp.int32), W1, b1, Wl, bl, Wr, Wn)


# speedup vs baseline: 1.4964x; 1.0227x over previous
import functools

import jax
import jax.numpy as jnp
from jax import lax
from jax.experimental import pallas as pl
from jax.experimental.pallas import tpu as pltpu
from jax.experimental.pallas import tpu_sc as plsc

_N = 10000
_E = 320000
_XD = 128
_HID = 128
_NCLS = 40

_NC = 2
_NS = 16
_NW = _NC * _NS
_K = 256
_T = 40
_NPAD = _N + 8

_RPS = _N // _NS
_KF = 80
_EPW = _E // _NW
_NCHF = _EPW // _KF
_ZCH = 125
_RBIG = 640
_RLAST = _N - 15 * _RBIG

_BN = 1000


def _worker_chunks(wid):
    qw = 39 * wid + jnp.minimum(wid, 2)
    cw = jnp.where(wid < 2, 40, 39)
    return qw, cw


def _fill(buf, val):
    v = jnp.full((16,), val, buf.dtype)

    @pl.loop(0, _K, step=16)
    def _(g):
        buf[pl.ds(g, 16)] = v


def _pre_body(x_ref, w1_ref, b1_ref, feat_ref, hp_ref):
    h = jnp.dot(x_ref[...], w1_ref[...], preferred_element_type=jnp.float32)
    h = h + b1_ref[...]
    feat_ref[...] = h
    hp_ref[...] = jnp.maximum(h, 0.0)


def _sc_cnt_body(e_hbm, src_out, dst_out, cnt_out,
                 ones_v, zbuf, pbuf, pbuf2, ib0, ib1, ib2, acc_sh,
                 i0, i1, i2, s0, s1, s2, psem):
    c = lax.axis_index("c")
    s = lax.axis_index("s")
    wid = c * _NS + s
    qw, cw = _worker_chunks(wid)
    base = pl.multiple_of(qw * _K, 8)

    def pass_leg(n, inbound):
        for row, buf, out in ((0, pbuf, src_out), (1, pbuf2, dst_out)):
            if inbound:
                pltpu.async_copy(e_hbm.at[row, pl.ds(base, n * _K)],
                                 buf.at[pl.ds(0, n * _K)], psem)
            else:
                pltpu.async_copy(buf.at[pl.ds(0, n * _K)],
                                 out.at[pl.ds(base, n * _K)], psem)

    def pass_wait(n):
        for buf in (pbuf, pbuf2):
            pltpu.make_async_copy(e_hbm.at[0, pl.ds(0, n * _K)],
                                  buf.at[pl.ds(0, n * _K)], psem).wait()

    @pl.when(wid < 2)
    def _():
        pass_leg(40, True)

    @pl.when(wid >= 2)
    def _():
        pass_leg(39, True)

    ov = jnp.ones((16,), jnp.float32)
    zv = jnp.zeros((16,), jnp.float32)

    @pl.loop(0, _K)
    def _(r):
        ones_v[r, pl.ds(0, 16)] = ov

    @pl.loop(0, _ZCH)
    def _(r):
        zbuf[r, pl.ds(0, 16)] = zv

    @pl.when(wid < 2)
    def _():
        pass_wait(40)
        pass_leg(40, False)

    @pl.when(wid >= 2)
    def _():
        pass_wait(39)
        pass_leg(39, False)

    @pl.loop(0, _RPS, step=_ZCH)
    def _(r0):
        pltpu.sync_copy(zbuf, acc_sh.at[pl.ds(s * _RPS + r0, _ZCH), :])

    @pl.when(wid < 2)
    def _():
        pass_wait(40)

    @pl.when(wid >= 2)
    def _():
        pass_wait(39)

    plsc.subcore_barrier()

    ibs = (ib0, ib1, ib2)
    isems = (i0, i1, i2)
    ssems = (s0, s1, s2)

    def load_idx(t, ib, sem):
        off = pl.multiple_of((qw + jnp.minimum(t, _T - 1)) * _K, 8)
        pltpu.async_copy(e_hbm.at[1, pl.ds(off, _K)], ib, sem)

    def wait_idx(ib, sem):
        pltpu.make_async_copy(e_hbm.at[1, pl.ds(0, _K)], ib, sem).wait()

    def start_sc(ib, sem):
        pltpu.async_copy(ones_v, acc_sh.at[ib], sem, add=True)

    def wait_sc(sem):
        pltpu.make_async_copy(ones_v, acc_sh.at[pl.ds(0, _K), :], sem).wait()

    def step(t, sl, first=False, last=False):
        r, r2 = sl, (sl + 2) % 3
        wait_idx(ibs[r], isems[r])

        @pl.when(t >= cw)
        def _():
            _fill(ibs[r], _N)

        start_sc(ibs[r], ssems[r])
        if not first:
            wait_sc(ssems[r2])
        if not last:
            load_idx(t + 2, ibs[r2], isems[r2])

    load_idx(jnp.int32(0), ib0, i0)
    load_idx(jnp.int32(1), ib1, i1)
    step(jnp.int32(0), 0, first=True)

    @pl.loop(0, 12)
    def _(j3):
        t = 3 * j3 + 1
        step(t, 1)
        step(t + 1, 2)
        step(t + 2, 0)

    step(jnp.int32(37), 1)
    step(jnp.int32(38), 2)
    step(jnp.int32(39), 0, last=True)
    wait_idx(ibs[1], isems[1])
    wait_sc(ssems[0])

    plsc.subcore_barrier()

    pltpu.sync_copy(acc_sh.at[pl.ds(s * _RPS, _RPS), :],
                    cnt_out.at[c, pl.ds(s * _RPS, _RPS), :])


def _sc_agg_body(src_hbm, dst_hbm, hp_hbm, out_hbm,
                 sb0, sb1, sb2, db0, db1, db2, rows0, rows1, rows2, acc_sh,
                 i0, i1, i2, g0, g1, g2, s0, s1, s2):
    c = lax.axis_index("c")
    s = lax.axis_index("s")
    wid = c * _NS + s
    base0 = wid * _EPW

    row0 = s * _RBIG
    zv = jnp.zeros((16,), jnp.float32)

    @pl.loop(0, _KF)
    def _(r):
        @pl.loop(0, _HID, step=16)
        def _(c0):
            rows0[r, pl.ds(c0, 16)] = zv

    @pl.when(s < 15)
    def _():
        @pl.loop(0, _RBIG, step=_KF)
        def _(r0):
            pltpu.sync_copy(rows0, acc_sh.at[pl.ds(row0 + r0, _KF), :])

    @pl.when(s == 15)
    def _():
        @pl.loop(0, _RLAST, step=_KF)
        def _(r0):
            pltpu.sync_copy(rows0, acc_sh.at[pl.ds(row0 + r0, _KF), :])

    plsc.subcore_barrier()

    sbs = (sb0, sb1, sb2)
    dbs = (db0, db1, db2)
    bufs = (rows0, rows1, rows2)
    isems = (i0, i1, i2)
    gsems = (g0, g1, g2)
    ssems = (s0, s1, s2)

    def load_idx(t, sb, db, sem):
        off = pl.multiple_of(base0 + jnp.minimum(t, _NCHF - 1) * _KF, 8)
        pltpu.async_copy(src_hbm.at[pl.ds(off, _KF)], sb, sem)
        pltpu.async_copy(dst_hbm.at[pl.ds(off, _KF)], db, sem)

    def wait_idx(sb, db, sem):
        pltpu.make_async_copy(src_hbm.at[pl.ds(0, _KF)], sb, sem).wait()
        pltpu.make_async_copy(dst_hbm.at[pl.ds(0, _KF)], db, sem).wait()

    def start_g(sb, buf, sem):
        pltpu.async_copy(hp_hbm.at[sb], buf, sem)

    def wait_g(buf, sem):
        pltpu.make_async_copy(hp_hbm.at[pl.ds(0, _KF), :], buf, sem).wait()

    def start_sc(buf, db, sem):
        pltpu.async_copy(buf, acc_sh.at[db], sem, add=True)

    def wait_sc(buf, sem):
        pltpu.make_async_copy(buf, acc_sh.at[pl.ds(0, _KF), :], sem).wait()

    def step(t, sl, first=False, last=False):
        r, r1, r2 = sl, (sl + 1) % 3, (sl + 2) % 3
        if not last:
            wait_idx(sbs[r1], dbs[r1], isems[r1])
        if not first:
            wait_sc(bufs[r2], ssems[r2])
        wait_g(bufs[r], gsems[r])
        if not last:
            start_g(sbs[r1], bufs[r1], gsems[r1])
        start_sc(bufs[r], dbs[r], ssems[r])
        if not last:
            load_idx(t + 2, sbs[r2], dbs[r2], isems[r2])

    load_idx(0, sb0, db0, i0)
    wait_idx(sb0, db0, i0)
    load_idx(1, sb1, db1, i1)
    start_g(sb0, rows0, g0)
    step(jnp.int32(0), 0, first=True)

    @pl.loop(0, (_NCHF - 2) // 3)
    def _(j3):
        t = 3 * j3 + 1
        step(t, 1)
        step(t + 1, 2)
        step(t + 2, 0)

    wait_idx(sbs[2], dbs[2], isems[2])
    wait_sc(bufs[0], ssems[0])
    wait_g(bufs[1], gsems[1])
    start_sc(bufs[1], dbs[1], ssems[1])
    wait_sc(bufs[1], ssems[1])

    plsc.subcore_barrier()

    @pl.when(s < 15)
    def _():
        pltpu.sync_copy(acc_sh.at[pl.ds(row0, _RBIG), :],
                        out_hbm.at[c, pl.ds(row0, _RBIG), :])

    @pl.when(s == 15)
    def _():
        pltpu.sync_copy(acc_sh.at[pl.ds(row0, _RLAST), :],
                        out_hbm.at[c, pl.ds(row0, _RLAST), :])


def _root_body(hp_ref, wr_ref, bl_ref, z_ref):
    z_ref[...] = (jnp.dot(hp_ref[...], wr_ref[...],
                          preferred_element_type=jnp.float32) + bl_ref[...])


def _post_body(part_ref, cnt_ref, z_ref, wl_ref, wn_ref,
               out_ref, of_ref):
    agg = part_ref[0] + part_ref[1]
    cnt = cnt_ref[0, :, :1] + cnt_ref[1, :, :1]
    mean = agg / jnp.maximum(cnt, 1.0)
    of = jnp.dot(mean, wl_ref[...], preferred_element_type=jnp.float32)
    of = of + z_ref[...]
    of_ref[...] = of
    nrm = jnp.sqrt(jnp.sum(of * of, axis=1, keepdims=True))
    xn = of / jnp.maximum(nrm, 1e-12)
    w = wn_ref[...]
    wnrm = jnp.sqrt(jnp.sum(w * w, axis=0, keepdims=True))
    wn = w / jnp.maximum(wnrm, 1e-12)
    out_ref[...] = 10.0 * jnp.dot(xn, wn, preferred_element_type=jnp.float32)


@jax.jit
def _run(x, e, W1, b1, Wl, bl, Wr, Wn):
    mesh = plsc.VectorSubcoreMesh(core_axis_name="c", subcore_axis_name="s")

    src1d, dst1d, cnt = pl.kernel(
        _sc_cnt_body,
        out_type=[
            jax.ShapeDtypeStruct((_E,), jnp.int32),
            jax.ShapeDtypeStruct((_E,), jnp.int32),
            jax.ShapeDtypeStruct((_NC, _N, 16), jnp.float32),
        ],
        mesh=mesh,
        compiler_params=pltpu.CompilerParams(use_tc_tiling_on_sc=False),
        scratch_types=[
            pltpu.VMEM((_K, 16), jnp.float32),
            pltpu.VMEM((_ZCH, 16), jnp.float32),
            pltpu.VMEM((40 * _K,), jnp.int32),
            pltpu.VMEM((40 * _K,), jnp.int32),
            pltpu.VMEM((_K,), jnp.int32),
            pltpu.VMEM((_K,), jnp.int32),
            pltpu.VMEM((_K,), jnp.int32),
            pltpu.VMEM_SHARED((_NPAD, 16), jnp.float32),
        ] + [pltpu.SemaphoreType.DMA] * 7,
    )(e)

    feat, hp = pl.pallas_call(
        _pre_body,
        grid=(_N // _BN,),
        in_specs=[
            pl.BlockSpec((_BN, _XD), lambda i: (i, 0)),
            pl.BlockSpec((_XD, _HID), lambda i: (0, 0)),
            pl.BlockSpec((1, _HID), lambda i: (0, 0)),
        ],
        out_specs=[
            pl.BlockSpec((_BN, _HID), lambda i: (i, 0)),
            pl.BlockSpec((_BN, _HID), lambda i: (i, 0)),
        ],
        out_shape=[
            jax.ShapeDtypeStruct((_N, _HID), jnp.float32),
            jax.ShapeDtypeStruct((_N, _HID), jnp.float32),
        ],
    )(x, W1, b1.reshape(1, _HID))

    partials = pl.kernel(
        _sc_agg_body,
        out_type=jax.ShapeDtypeStruct((_NC, _N, _HID), jnp.float32),
        mesh=mesh,
        compiler_params=pltpu.CompilerParams(use_tc_tiling_on_sc=True),
        scratch_types=[
            pltpu.VMEM((_KF,), jnp.int32),
            pltpu.VMEM((_KF,), jnp.int32),
            pltpu.VMEM((_KF,), jnp.int32),
            pltpu.VMEM((_KF,), jnp.int32),
            pltpu.VMEM((_KF,), jnp.int32),
            pltpu.VMEM((_KF,), jnp.int32),
            pltpu.VMEM((_KF, _HID), jnp.float32),
            pltpu.VMEM((_KF, _HID), jnp.float32),
            pltpu.VMEM((_KF, _HID), jnp.float32),
            pltpu.VMEM_SHARED((_N, _HID), jnp.float32),
        ] + [pltpu.SemaphoreType.DMA] * 9,
    )(src1d, dst1d, hp)

    z = pl.pallas_call(
        _root_body,
        grid=(_N // _BN,),
        in_specs=[
            pl.BlockSpec((_BN, _HID), lambda i: (i, 0)),
            pl.BlockSpec((_HID, _HID), lambda i: (0, 0)),
            pl.BlockSpec((1, _HID), lambda i: (0, 0)),
        ],
        out_specs=pl.BlockSpec((_BN, _HID), lambda i: (i, 0)),
        out_shape=jax.ShapeDtypeStruct((_N, _HID), jnp.float32),
    )(hp, Wr, bl.reshape(1, _HID))

    out, out_feat = pl.pallas_call(
        _post_body,
        grid=(_N // _BN,),
        in_specs=[
            pl.BlockSpec((_NC, _BN, _HID), lambda i: (0, i, 0)),
            pl.BlockSpec((_NC, _BN, 16), lambda i: (0, i, 0)),
            pl.BlockSpec((_BN, _HID), lambda i: (i, 0)),
            pl.BlockSpec((_HID, _HID), lambda i: (0, 0)),
            pl.BlockSpec((_HID, _NCLS), lambda i: (0, 0)),
        ],
        out_specs=[
            pl.BlockSpec((_BN, _NCLS), lambda i: (i, 0)),
            pl.BlockSpec((_BN, _HID), lambda i: (i, 0)),
        ],
        out_shape=[
            jax.ShapeDtypeStruct((_N, _NCLS), jnp.float32),
            jax.ShapeDtypeStruct((_N, _HID), jnp.float32),
        ],
    )(partials, cnt, z, Wl, Wn)

    return out, feat, out_feat


def kernel(x, edge_index, W1, b1, Wl, bl, Wr, Wn):
    return _run(x, edge_index.astype(jnp.int32), W1, b1, Wl, bl, Wr, Wn)
